# baseline (speedup 1.0000x reference)
import functools

import jax
import jax.numpy as jnp
from jax import lax
from jax.experimental import pallas as pl
from jax.experimental.pallas import tpu as pltpu
from jax.experimental.pallas import tpu_sc as plsc

N_NODES = 10000
D = 128
N_PAD = 10112
E_RAW = 320000
E_TOT = E_RAW + N_NODES
NW = 32
NS = 16
B1_CHUNK = 128
CHUNKS_B1 = 82
E_PAD = NW * CHUNKS_B1 * B1_CHUNK
EDGES_PER_TILE = CHUNKS_B1 * B1_CHUNK
FULL_LAST = (E_TOT - (NW - 1) * EDGES_PER_TILE) // B1_CHUNK
TAIL_ROWS = E_TOT - (NW - 1) * EDGES_PER_TILE - FULL_LAST * B1_CHUNK
CHUNK_B2 = 4096
N_CHUNKS_B2 = E_PAD // CHUNK_B2
F_PER_TILE = D // NW


def _tc_a_body(x_ref, wt_ref, ap_ref, act_ref, al_ref):
    act = jnp.dot(x_ref[...], wt_ref[...], preferred_element_type=jnp.float32)
    act_ref[...] = act
    al_ref[...] = lax.dot_general(
        ap_ref[...], act, (((1,), (1,)), ((), ())),
        preferred_element_type=jnp.float32,
        precision=lax.Precision.HIGHEST,
    )


def _tc_a(x_pad, w_t, a_pair):
    nblk = N_PAD // 128
    return pl.pallas_call(
        _tc_a_body,
        grid=(nblk,),
        in_specs=[
            pl.BlockSpec((128, D), lambda i: (i, 0)),
            pl.BlockSpec((D, D), lambda i: (0, 0)),
            pl.BlockSpec((2, D), lambda i: (0, 0)),
        ],
        out_specs=[
            pl.BlockSpec((128, D), lambda i: (i, 0)),
            pl.BlockSpec((2, 128), lambda i: (0, i)),
        ],
        out_shape=[
            jax.ShapeDtypeStruct((N_PAD, D), jnp.float32),
            jax.ShapeDtypeStruct((2, N_PAD), jnp.float32),
        ],
    )(x_pad, w_t, a_pair)


def _make_b1():
    mesh = plsc.VectorSubcoreMesh(
        core_axis_name="c", subcore_axis_name="s", num_cores=2, num_subcores=16
    )
    rows_per_sub = N_PAD // NS

    @functools.partial(
        pl.kernel,
        mesh=mesh,
        compiler_params=pltpu.CompilerParams(
            needs_layout_passes=False, use_tc_tiling_on_sc=False
        ),
        out_type=[
            jax.ShapeDtypeStruct((E_TOT, D), jnp.float32),
            jax.ShapeDtypeStruct((NW, CHUNKS_B1, B1_CHUNK), jnp.float32),
            jax.ShapeDtypeStruct((EDGES_PER_TILE, D), jnp.float32),
        ],
        scratch_types=[
            pltpu.VMEM((CHUNKS_B1, B1_CHUNK), jnp.int32),
            pltpu.VMEM((CHUNKS_B1, B1_CHUNK), jnp.int32),
            pltpu.VMEM((N_PAD,), jnp.float32),
            pltpu.VMEM((N_PAD,), jnp.float32),
            pltpu.VMEM((CHUNKS_B1, B1_CHUNK), jnp.float32),
            pltpu.VMEM((4, B1_CHUNK, D), jnp.float32),
            pltpu.SemaphoreType.DMA,
            pltpu.SemaphoreType.DMA,
            pltpu.SemaphoreType.DMA,
            pltpu.SemaphoreType.DMA,
        ],
    )
    def b1(act_hbm, ad_hbm, as_hbm, src_hbm, dst_hbm, msgs_hbm, w_hbm,
           sink_hbm, src_v, dst_v, ad_v, as_v, w_v, rows_v,
           gsem0, gsem1, wsem0, wsem1):
        sid = lax.axis_index("s")
        wid = sid * 2 + lax.axis_index("c")
        base = wid * EDGES_PER_TILE
        pltpu.sync_copy(src_hbm.at[wid], src_v)
        pltpu.sync_copy(dst_hbm.at[wid], dst_v)
        pltpu.sync_copy(ad_hbm, ad_v)
        pltpu.sync_copy(as_hbm, as_v)
        pltpu.async_copy(act_hbm.at[src_v.at[0]], rows_v.at[0], gsem0)
        pltpu.async_copy(act_hbm.at[src_v.at[1]], rows_v.at[1], gsem1)

        def step(c, buf, buf2, gsem, wsem):
            pltpu.make_async_copy(
                act_hbm.at[src_v.at[c]], rows_v.at[buf], gsem
            ).wait()

            @pl.when(c >= 2)
            def _():
                pltpu.make_async_copy(
                    rows_v.at[buf], sink_hbm.at[pl.ds(0, B1_CHUNK)], wsem
                ).wait()

            full = jnp.logical_or(wid < NW - 1, c < FULL_LAST)

            @pl.when(full)
            def _():
                pltpu.async_copy(
                    rows_v.at[buf],
                    msgs_hbm.at[pl.ds(base + c * B1_CHUNK, B1_CHUNK)],
                    wsem,
                )

            @pl.when(jnp.logical_not(full))
            def _():
                pltpu.async_copy(
                    rows_v.at[buf],
                    sink_hbm.at[pl.ds(c * B1_CHUNK, B1_CHUNK)],
                    wsem,
                )

            @pl.when(jnp.logical_and(wid == NW - 1, c == FULL_LAST))
            def _():
                pltpu.sync_copy(
                    rows_v.at[buf, pl.ds(0, TAIL_ROWS)],
                    msgs_hbm.at[pl.ds(base + c * B1_CHUNK, TAIL_ROWS)],
                )

            @pl.when(c + 2 < CHUNKS_B1)
            def _():
                pltpu.async_copy(
                    act_hbm.at[src_v.at[c + 2]], rows_v.at[buf2], gsem
                )

            for g in range(B1_CHUNK // 16):
                s16 = src_v[c, pl.ds(g * 16, 16)]
                d16 = dst_v[c, pl.ds(g * 16, 16)]
                t = plsc.load_gather(ad_v, [d16]) + plsc.load_gather(as_v, [s16])
                w_v[c, pl.ds(g * 16, 16)] = jnp.exp(jnp.maximum(t, 0.01 * t))

        def body(cc, carry):
            half = lax.rem(cc, 2) * 2
            half2 = lax.rem(cc + 1, 2) * 2
            step(cc * 2, half, half2, gsem0, wsem0)
            step(cc * 2 + 1, half + 1, half2 + 1, gsem1, wsem1)
            return carry

        lax.fori_loop(0, CHUNKS_B1 // 2, body, 0)
        pltpu.make_async_copy(
            rows_v.at[0], sink_hbm.at[pl.ds(0, B1_CHUNK)], wsem0
        ).wait()
        pltpu.make_async_copy(
            rows_v.at[0], sink_hbm.at[pl.ds(0, B1_CHUNK)], wsem1
        ).wait()
        pltpu.sync_copy(w_v, w_hbm.at[wid])

    return b1


def _make_b2():
    mesh = plsc.VectorSubcoreMesh(
        core_axis_name="c", subcore_axis_name="s", num_cores=2, num_subcores=16
    )
    n_grp = CHUNK_B2 // 16
    e_per_sub = E_PAD // NS

    @functools.partial(
        pl.kernel,
        mesh=mesh,
        compiler_params=pltpu.CompilerParams(
            needs_layout_passes=False, use_tc_tiling_on_sc=False
        ),
        out_type=[
            jax.ShapeDtypeStruct((NW, F_PER_TILE * N_PAD), jnp.float32),
            jax.ShapeDtypeStruct((N_PAD,), jnp.float32),
        ],
        scratch_types=[
            pltpu.VMEM((F_PER_TILE * N_PAD,), jnp.float32),
            pltpu.VMEM((F_PER_TILE * N_PAD,), jnp.float32),
            pltpu.VMEM((N_PAD,), jnp.float32),
            pltpu.VMEM((2, CHUNK_B2), jnp.int32),
            pltpu.VMEM((2, CHUNK_B2), jnp.float32),
            pltpu.SemaphoreType.DMA,
        ],
    )
    def b2(act_hbm, sd_hbm, w_hbm, agg_hbm, den_hbm,
           actc_v, agg_v, den_v, sd_v, wc_v, esem):
        sid = lax.axis_index("s")
        wid = sid * 2 + lax.axis_index("c")
        pltpu.sync_copy(act_hbm.at[wid], actc_v)

        zf = jnp.zeros((16,), jnp.float32)

        @plsc.parallel_loop(0, N_PAD // 16, unroll=8)
        def _zero(i):
            den_v[pl.ds(i * 16, 16)] = zf
            for f in range(F_PER_TILE):
                agg_v[pl.ds(f * N_PAD + i * 16, 16)] = zf

        def start_chunk(ci, buf):
            off = ci * CHUNK_B2
            pltpu.async_copy(sd_hbm.at[pl.ds(off, CHUNK_B2)], sd_v.at[buf], esem)
            pltpu.async_copy(w_hbm.at[pl.ds(off, CHUNK_B2)], wc_v.at[buf], esem)

        def wait_chunk(ci, buf):
            off = ci * CHUNK_B2
            pltpu.make_async_copy(
                sd_hbm.at[pl.ds(off, CHUNK_B2)], sd_v.at[buf], esem
            ).wait()
            pltpu.make_async_copy(
                w_hbm.at[pl.ds(off, CHUNK_B2)], wc_v.at[buf], esem
            ).wait()

        start_chunk(0, 0)

        def chunk_body(ci, carry):
            buf = lax.rem(ci, 2)
            wait_chunk(ci, buf)

            @pl.when(ci + 1 < N_CHUNKS_B2)
            def _():
                start_chunk(ci + 1, 1 - buf)

            @plsc.parallel_loop(0, n_grp, unroll=4)
            def _grp(g):
                o = g * 16
                sd = sd_v[buf, pl.ds(o, 16)]
                w16 = wc_v[buf, pl.ds(o, 16)]
                s16 = jnp.bitwise_and(sd, 0xFFFF)
                d16 = jnp.right_shift(sd, 16)
                for f in range(F_PER_TILE):
                    av = plsc.load_gather(actc_v, [s16 + f * N_PAD] if f else [s16])
                    plsc.addupdate_scatter(
                        agg_v, [d16 + f * N_PAD] if f else [d16], av * w16
                    )
                plsc.addupdate_scatter(den_v, [d16], w16)

            return carry

        lax.fori_loop(0, N_CHUNKS_B2, chunk_body, 0)

        @plsc.parallel_loop(0, N_PAD // 16, unroll=8)
        def _div(i):
            o = i * 16
            dn = den_v[pl.ds(o, 16)]
            dn = jnp.where(dn == 0.0, 1.0, dn)
            den_v[pl.ds(o, 16)] = dn
            for f in range(F_PER_TILE):
                g = agg_v[pl.ds(f * N_PAD + o, 16)]
                agg_v[pl.ds(f * N_PAD + o, 16)] = g / dn

        pltpu.sync_copy(agg_v, agg_hbm.at[wid])

        @pl.when(wid == 0)
        def _():
            pltpu.sync_copy(den_v, den_hbm)

    return b2


def kernel(x, edge_index, W, a):
    x = x.astype(jnp.float32)
    x_pad = jnp.concatenate(
        [x, jnp.zeros((N_PAD - N_NODES, D), jnp.float32)], axis=0
    )
    w_t = W.astype(jnp.float32).T
    a_pair = a.astype(jnp.float32).reshape(2, D)

    loop_ids = jnp.arange(N_NODES, dtype=jnp.int32)
    pad_src = jnp.arange(E_PAD - E_TOT, dtype=jnp.int32) % N_NODES
    pad_dst = jnp.full((E_PAD - E_TOT,), N_NODES, jnp.int32)
    src = jnp.concatenate([edge_index[0].astype(jnp.int32), loop_ids, pad_src])
    dst = jnp.concatenate([edge_index[1].astype(jnp.int32), loop_ids, pad_dst])
    src3 = src.reshape(NW, CHUNKS_B1, B1_CHUNK)
    dst3 = dst.reshape(NW, CHUNKS_B1, B1_CHUNK)
    sd_packed = jnp.bitwise_or(jnp.left_shift(dst, 16), src)

    act, al = _tc_a(x_pad, w_t, a_pair)
    ad = al[0]
    as_ = al[1]

    msgs, w3, _sink = _make_b1()(act, ad, as_, src3, dst3)
    w_flat = w3.reshape(E_PAD)

    act_cols = act.T.reshape(NW, F_PER_TILE * N_PAD)
    agg_cols, den_pad = _make_b2()(act_cols, sd_packed, w_flat)
    agg_pad = agg_cols.reshape(D, N_PAD).T

    return (
        agg_pad[:N_NODES],
        w_flat[:E_TOT],
        den_pad[:N_NODES],
        msgs,
    )


# speedup vs baseline: 1.0842x; 1.0842x over previous
import functools

import jax
import jax.numpy as jnp
from jax import lax
from jax.experimental import pallas as pl
from jax.experimental.pallas import tpu as pltpu
from jax.experimental.pallas import tpu_sc as plsc

N_NODES = 10000
D = 128
N_PAD = 10112
E_RAW = 320000
E_TOT = E_RAW + N_NODES
NW = 32
NS = 16
B1_CHUNK = 128
CHUNKS_B1 = 82
E_PAD = NW * CHUNKS_B1 * B1_CHUNK
EDGES_PER_TILE = CHUNKS_B1 * B1_CHUNK
FULL_LAST = (E_TOT - (NW - 1) * EDGES_PER_TILE) // B1_CHUNK
TAIL_ROWS = E_TOT - (NW - 1) * EDGES_PER_TILE - FULL_LAST * B1_CHUNK
CHUNK_B2 = 4096
N_CHUNKS_B2 = E_PAD // CHUNK_B2
F_PER_TILE = D // NW


def _tc_a_body(x_ref, wt_ref, ap_ref, act_ref, al_ref):
    act = jnp.dot(x_ref[...], wt_ref[...], preferred_element_type=jnp.float32)
    act_ref[...] = act
    al_ref[...] = lax.dot_general(
        ap_ref[...], act, (((1,), (1,)), ((), ())),
        preferred_element_type=jnp.float32,
        precision=lax.Precision.HIGHEST,
    )


def _tc_a(x_pad, w_t, a_pair):
    blk = N_PAD
    nblk = 1
    return pl.pallas_call(
        _tc_a_body,
        grid=(nblk,),
        in_specs=[
            pl.BlockSpec((blk, D), lambda i: (i, 0)),
            pl.BlockSpec((D, D), lambda i: (0, 0)),
            pl.BlockSpec((2, D), lambda i: (0, 0)),
        ],
        out_specs=[
            pl.BlockSpec((blk, D), lambda i: (i, 0)),
            pl.BlockSpec((2, blk), lambda i: (0, i)),
        ],
        out_shape=[
            jax.ShapeDtypeStruct((N_PAD, D), jnp.float32),
            jax.ShapeDtypeStruct((2, N_PAD), jnp.float32),
        ],
    )(x_pad, w_t, a_pair)


def _make_b1():
    mesh = plsc.VectorSubcoreMesh(
        core_axis_name="c", subcore_axis_name="s", num_cores=2, num_subcores=16
    )
    rows_per_sub = N_PAD // NS

    @functools.partial(
        pl.kernel,
        mesh=mesh,
        compiler_params=pltpu.CompilerParams(
            needs_layout_passes=False, use_tc_tiling_on_sc=False
        ),
        out_type=[
            jax.ShapeDtypeStruct((E_TOT, D), jnp.float32),
            jax.ShapeDtypeStruct((NW, CHUNKS_B1, B1_CHUNK), jnp.float32),
            jax.ShapeDtypeStruct((EDGES_PER_TILE, D), jnp.float32),
        ],
        scratch_types=[
            pltpu.VMEM((CHUNKS_B1, B1_CHUNK), jnp.int32),
            pltpu.VMEM((CHUNKS_B1, B1_CHUNK), jnp.int32),
            pltpu.VMEM((N_PAD,), jnp.float32),
            pltpu.VMEM((N_PAD,), jnp.float32),
            pltpu.VMEM((CHUNKS_B1, B1_CHUNK), jnp.float32),
            pltpu.VMEM((4, B1_CHUNK, D), jnp.float32),
            pltpu.SemaphoreType.DMA,
            pltpu.SemaphoreType.DMA,
            pltpu.SemaphoreType.DMA,
            pltpu.SemaphoreType.DMA,
        ],
    )
    def b1(act_hbm, ad_hbm, as_hbm, src_hbm, dst_hbm, msgs_hbm, w_hbm,
           sink_hbm, src_v, dst_v, ad_v, as_v, w_v, rows_v,
           gsem0, gsem1, wsem0, wsem1):
        sid = lax.axis_index("s")
        wid = sid * 2 + lax.axis_index("c")
        base = wid * EDGES_PER_TILE
        pltpu.sync_copy(src_hbm.at[wid], src_v)
        pltpu.sync_copy(dst_hbm.at[wid], dst_v)
        pltpu.sync_copy(ad_hbm, ad_v)
        pltpu.sync_copy(as_hbm, as_v)
        pltpu.async_copy(act_hbm.at[src_v.at[0]], rows_v.at[0], gsem0)
        pltpu.async_copy(act_hbm.at[src_v.at[1]], rows_v.at[1], gsem1)

        def step(c, buf, buf2, gsem, wsem):
            pltpu.make_async_copy(
                act_hbm.at[src_v.at[c]], rows_v.at[buf], gsem
            ).wait()

            @pl.when(c >= 2)
            def _():
                pltpu.make_async_copy(
                    rows_v.at[buf], sink_hbm.at[pl.ds(0, B1_CHUNK)], wsem
                ).wait()

            full = jnp.logical_or(wid < NW - 1, c < FULL_LAST)

            @pl.when(full)
            def _():
                pltpu.async_copy(
                    rows_v.at[buf],
                    msgs_hbm.at[pl.ds(base + c * B1_CHUNK, B1_CHUNK)],
                    wsem,
                )

            @pl.when(jnp.logical_not(full))
            def _():
                pltpu.async_copy(
                    rows_v.at[buf],
                    sink_hbm.at[pl.ds(c * B1_CHUNK, B1_CHUNK)],
                    wsem,
                )

            @pl.when(jnp.logical_and(wid == NW - 1, c == FULL_LAST))
            def _():
                pltpu.sync_copy(
                    rows_v.at[buf, pl.ds(0, TAIL_ROWS)],
                    msgs_hbm.at[pl.ds(base + c * B1_CHUNK, TAIL_ROWS)],
                )

            @pl.when(c + 2 < CHUNKS_B1)
            def _():
                pltpu.async_copy(
                    act_hbm.at[src_v.at[c + 2]], rows_v.at[buf2], gsem
                )

            for g in range(B1_CHUNK // 16):
                s16 = src_v[c, pl.ds(g * 16, 16)]
                d16 = dst_v[c, pl.ds(g * 16, 16)]
                t = plsc.load_gather(ad_v, [d16]) + plsc.load_gather(as_v, [s16])
                w_v[c, pl.ds(g * 16, 16)] = jnp.exp(jnp.maximum(t, 0.01 * t))

        def body(cc, carry):
            half = lax.rem(cc, 2) * 2
            half2 = lax.rem(cc + 1, 2) * 2
            step(cc * 2, half, half2, gsem0, wsem0)
            step(cc * 2 + 1, half + 1, half2 + 1, gsem1, wsem1)
            return carry

        lax.fori_loop(0, CHUNKS_B1 // 2, body, 0)
        pltpu.make_async_copy(
            rows_v.at[0], sink_hbm.at[pl.ds(0, B1_CHUNK)], wsem0
        ).wait()
        pltpu.make_async_copy(
            rows_v.at[0], sink_hbm.at[pl.ds(0, B1_CHUNK)], wsem1
        ).wait()
        pltpu.sync_copy(w_v, w_hbm.at[wid])

    return b1


def _make_b2():
    mesh = plsc.VectorSubcoreMesh(
        core_axis_name="c", subcore_axis_name="s", num_cores=2, num_subcores=16
    )
    n_grp = CHUNK_B2 // 16
    e_per_sub = E_PAD // NS

    @functools.partial(
        pl.kernel,
        mesh=mesh,
        compiler_params=pltpu.CompilerParams(
            needs_layout_passes=False, use_tc_tiling_on_sc=False
        ),
        out_type=[
            jax.ShapeDtypeStruct((NW, F_PER_TILE * N_PAD), jnp.float32),
            jax.ShapeDtypeStruct((N_PAD,), jnp.float32),
        ],
        scratch_types=[
            pltpu.VMEM((F_PER_TILE * N_PAD,), jnp.float32),
            pltpu.VMEM((F_PER_TILE * N_PAD,), jnp.float32),
            pltpu.VMEM((N_PAD,), jnp.float32),
            pltpu.VMEM((2, CHUNK_B2), jnp.int32),
            pltpu.VMEM((2, CHUNK_B2), jnp.float32),
            pltpu.SemaphoreType.DMA,
        ],
    )
    def b2(act_hbm, sd_hbm, w_hbm, agg_hbm, den_hbm,
           actc_v, agg_v, den_v, sd_v, wc_v, esem):
        sid = lax.axis_index("s")
        wid = sid * 2 + lax.axis_index("c")
        pltpu.sync_copy(act_hbm.at[wid], actc_v)

        zf = jnp.zeros((16,), jnp.float32)

        @plsc.parallel_loop(0, N_PAD // 16, unroll=8)
        def _zero(i):
            den_v[pl.ds(i * 16, 16)] = zf
            for f in range(F_PER_TILE):
                agg_v[pl.ds(f * N_PAD + i * 16, 16)] = zf

        def start_chunk(ci, buf):
            off = ci * CHUNK_B2
            pltpu.async_copy(sd_hbm.at[pl.ds(off, CHUNK_B2)], sd_v.at[buf], esem)
            pltpu.async_copy(w_hbm.at[pl.ds(off, CHUNK_B2)], wc_v.at[buf], esem)

        def wait_chunk(ci, buf):
            off = ci * CHUNK_B2
            pltpu.make_async_copy(
                sd_hbm.at[pl.ds(off, CHUNK_B2)], sd_v.at[buf], esem
            ).wait()
            pltpu.make_async_copy(
                w_hbm.at[pl.ds(off, CHUNK_B2)], wc_v.at[buf], esem
            ).wait()

        start_chunk(0, 0)

        def chunk_body(ci, carry):
            buf = lax.rem(ci, 2)
            wait_chunk(ci, buf)

            @pl.when(ci + 1 < N_CHUNKS_B2)
            def _():
                start_chunk(ci + 1, 1 - buf)

            @plsc.parallel_loop(0, n_grp, unroll=8)
            def _grp(g):
                o = g * 16
                sd = sd_v[buf, pl.ds(o, 16)]
                w16 = wc_v[buf, pl.ds(o, 16)]
                s16 = jnp.bitwise_and(sd, 0xFFFF)
                d16 = jnp.right_shift(sd, 16)
                for f in range(F_PER_TILE):
                    av = plsc.load_gather(actc_v, [s16 + f * N_PAD] if f else [s16])
                    plsc.addupdate_scatter(
                        agg_v, [d16 + f * N_PAD] if f else [d16], av * w16
                    )
                plsc.addupdate_scatter(den_v, [d16], w16)

            return carry

        lax.fori_loop(0, N_CHUNKS_B2, chunk_body, 0)

        @plsc.parallel_loop(0, N_PAD // 16, unroll=8)
        def _div(i):
            o = i * 16
            dn = den_v[pl.ds(o, 16)]
            dn = jnp.where(dn == 0.0, 1.0, dn)
            den_v[pl.ds(o, 16)] = dn
            for f in range(F_PER_TILE):
                g = agg_v[pl.ds(f * N_PAD + o, 16)]
                agg_v[pl.ds(f * N_PAD + o, 16)] = g / dn

        pltpu.sync_copy(agg_v, agg_hbm.at[wid])

        @pl.when(wid == 0)
        def _():
            pltpu.sync_copy(den_v, den_hbm)

    return b2


def kernel(x, edge_index, W, a):
    x = x.astype(jnp.float32)
    x_pad = jnp.concatenate(
        [x, jnp.zeros((N_PAD - N_NODES, D), jnp.float32)], axis=0
    )
    w_t = W.astype(jnp.float32).T
    a_pair = a.astype(jnp.float32).reshape(2, D)

    loop_ids = jnp.arange(N_NODES, dtype=jnp.int32)
    pad_src = jnp.arange(E_PAD - E_TOT, dtype=jnp.int32) % N_NODES
    pad_dst = jnp.full((E_PAD - E_TOT,), N_NODES, jnp.int32)
    src = jnp.concatenate([edge_index[0].astype(jnp.int32), loop_ids, pad_src])
    dst = jnp.concatenate([edge_index[1].astype(jnp.int32), loop_ids, pad_dst])
    src3 = src.reshape(NW, CHUNKS_B1, B1_CHUNK)
    dst3 = dst.reshape(NW, CHUNKS_B1, B1_CHUNK)
    sd_packed = jnp.bitwise_or(jnp.left_shift(dst, 16), src)

    act, al = _tc_a(x_pad, w_t, a_pair)
    ad = al[0]
    as_ = al[1]

    msgs, w3, _sink = _make_b1()(act, ad, as_, src3, dst3)
    w_flat = w3.reshape(E_PAD)

    act_cols = act.T.reshape(NW, F_PER_TILE * N_PAD)
    agg_cols, den_pad = _make_b2()(act_cols, sd_packed, w_flat)
    agg_pad = agg_cols.reshape(D, N_PAD).T

    return (
        agg_pad[:N_NODES],
        w_flat[:E_TOT],
        den_pad[:N_NODES],
        msgs,
    )


# speedup vs baseline: 1.1597x; 1.0696x over previous
import functools

import jax
import jax.numpy as jnp
from jax import lax
from jax.experimental import pallas as pl
from jax.experimental.pallas import tpu as pltpu
from jax.experimental.pallas import tpu_sc as plsc

N_NODES = 10000
D = 128
N_PAD = 10112
E_RAW = 320000
E_TOT = E_RAW + N_NODES
NW = 32
NS = 16
B1_CHUNK = 128
CHUNKS_B1 = 82
E_PAD = NW * CHUNKS_B1 * B1_CHUNK
EDGES_PER_TILE = CHUNKS_B1 * B1_CHUNK
FULL_LAST = (E_TOT - (NW - 1) * EDGES_PER_TILE) // B1_CHUNK
TAIL_ROWS = E_TOT - (NW - 1) * EDGES_PER_TILE - FULL_LAST * B1_CHUNK
CHUNK_B2 = 4096
N_CHUNKS_B2 = E_PAD // CHUNK_B2
F_PER_TILE = D // NW


def _tc_a_body(x_ref, wt_ref, ap_ref, act_ref, al_ref):
    act = jnp.dot(x_ref[...], wt_ref[...], preferred_element_type=jnp.float32)
    act_ref[...] = act
    al_ref[...] = lax.dot_general(
        ap_ref[...], act, (((1,), (1,)), ((), ())),
        preferred_element_type=jnp.float32,
        precision=lax.Precision.HIGHEST,
    )


def _tc_a(x_pad, w_t, a_pair):
    blk = N_PAD
    nblk = 1
    return pl.pallas_call(
        _tc_a_body,
        grid=(nblk,),
        in_specs=[
            pl.BlockSpec((blk, D), lambda i: (i, 0)),
            pl.BlockSpec((D, D), lambda i: (0, 0)),
            pl.BlockSpec((2, D), lambda i: (0, 0)),
        ],
        out_specs=[
            pl.BlockSpec((blk, D), lambda i: (i, 0)),
            pl.BlockSpec((2, blk), lambda i: (0, i)),
        ],
        out_shape=[
            jax.ShapeDtypeStruct((N_PAD, D), jnp.float32),
            jax.ShapeDtypeStruct((2, N_PAD), jnp.float32),
        ],
    )(x_pad, w_t, a_pair)


def _make_b1():
    mesh = plsc.VectorSubcoreMesh(
        core_axis_name="c", subcore_axis_name="s", num_cores=2, num_subcores=16
    )
    rows_per_sub = N_PAD // NS

    @functools.partial(
        pl.kernel,
        mesh=mesh,
        compiler_params=pltpu.CompilerParams(
            needs_layout_passes=False, use_tc_tiling_on_sc=False
        ),
        out_type=[
            jax.ShapeDtypeStruct((E_TOT, D), jnp.float32),
            jax.ShapeDtypeStruct((NW, CHUNKS_B1, B1_CHUNK), jnp.float32),
            jax.ShapeDtypeStruct((EDGES_PER_TILE, D), jnp.float32),
            jax.ShapeDtypeStruct((NW, N_PAD), jnp.float32),
        ],
        scratch_types=[
            pltpu.VMEM((CHUNKS_B1, B1_CHUNK), jnp.int32),
            pltpu.VMEM((CHUNKS_B1, B1_CHUNK), jnp.int32),
            pltpu.VMEM((N_PAD,), jnp.float32),
            pltpu.VMEM((N_PAD,), jnp.float32),
            pltpu.VMEM((CHUNKS_B1, B1_CHUNK), jnp.float32),
            pltpu.VMEM((N_PAD,), jnp.float32),
            pltpu.VMEM((4, B1_CHUNK, D), jnp.float32),
            pltpu.SemaphoreType.DMA,
            pltpu.SemaphoreType.DMA,
            pltpu.SemaphoreType.DMA,
            pltpu.SemaphoreType.DMA,
        ],
    )
    def b1(act_hbm, ad_hbm, as_hbm, src_hbm, dst_hbm, msgs_hbm, w_hbm,
           sink_hbm, denp_hbm, src_v, dst_v, ad_v, as_v, w_v, den_v, rows_v,
           gsem0, gsem1, wsem0, wsem1):
        sid = lax.axis_index("s")
        wid = sid * 2 + lax.axis_index("c")
        base = wid * EDGES_PER_TILE
        pltpu.sync_copy(src_hbm.at[wid], src_v)
        pltpu.sync_copy(dst_hbm.at[wid], dst_v)
        pltpu.sync_copy(ad_hbm, ad_v)
        pltpu.sync_copy(as_hbm, as_v)

        zf = jnp.zeros((16,), jnp.float32)

        @plsc.parallel_loop(0, N_PAD // 16, unroll=8)
        def _zero(i):
            den_v[pl.ds(i * 16, 16)] = zf

        pltpu.async_copy(act_hbm.at[src_v.at[0]], rows_v.at[0], gsem0)
        pltpu.async_copy(act_hbm.at[src_v.at[1]], rows_v.at[1], gsem1)

        def step(c, buf, buf2, gsem, wsem):
            pltpu.make_async_copy(
                act_hbm.at[src_v.at[c]], rows_v.at[buf], gsem
            ).wait()

            @pl.when(c >= 2)
            def _():
                pltpu.make_async_copy(
                    rows_v.at[buf], sink_hbm.at[pl.ds(0, B1_CHUNK)], wsem
                ).wait()

            full = jnp.logical_or(wid < NW - 1, c < FULL_LAST)

            @pl.when(full)
            def _():
                pltpu.async_copy(
                    rows_v.at[buf],
                    msgs_hbm.at[pl.ds(base + c * B1_CHUNK, B1_CHUNK)],
                    wsem,
                )

            @pl.when(jnp.logical_not(full))
            def _():
                pltpu.async_copy(
                    rows_v.at[buf],
                    sink_hbm.at[pl.ds(c * B1_CHUNK, B1_CHUNK)],
                    wsem,
                )

            @pl.when(jnp.logical_and(wid == NW - 1, c == FULL_LAST))
            def _():
                pltpu.sync_copy(
                    rows_v.at[buf, pl.ds(0, TAIL_ROWS)],
                    msgs_hbm.at[pl.ds(base + c * B1_CHUNK, TAIL_ROWS)],
                )

            @pl.when(c + 2 < CHUNKS_B1)
            def _():
                pltpu.async_copy(
                    act_hbm.at[src_v.at[c + 2]], rows_v.at[buf2], gsem
                )

            for g in range(B1_CHUNK // 16):
                s16 = src_v[c, pl.ds(g * 16, 16)]
                d16 = dst_v[c, pl.ds(g * 16, 16)]
                t = plsc.load_gather(ad_v, [d16]) + plsc.load_gather(as_v, [s16])
                w16 = jnp.exp(jnp.maximum(t, 0.01 * t))
                w_v[c, pl.ds(g * 16, 16)] = w16
                plsc.addupdate_scatter(den_v, [d16], w16)

        def body(cc, carry):
            half = lax.rem(cc, 2) * 2
            half2 = lax.rem(cc + 1, 2) * 2
            step(cc * 2, half, half2, gsem0, wsem0)
            step(cc * 2 + 1, half + 1, half2 + 1, gsem1, wsem1)
            return carry

        lax.fori_loop(0, CHUNKS_B1 // 2, body, 0)
        pltpu.make_async_copy(
            rows_v.at[0], sink_hbm.at[pl.ds(0, B1_CHUNK)], wsem0
        ).wait()
        pltpu.make_async_copy(
            rows_v.at[0], sink_hbm.at[pl.ds(0, B1_CHUNK)], wsem1
        ).wait()
        pltpu.sync_copy(w_v, w_hbm.at[wid])
        pltpu.sync_copy(den_v, denp_hbm.at[wid])

    return b1


def _make_b2():
    mesh = plsc.VectorSubcoreMesh(
        core_axis_name="c", subcore_axis_name="s", num_cores=2, num_subcores=16
    )
    n_grp = CHUNK_B2 // 16
    e_per_sub = E_PAD // NS

    @functools.partial(
        pl.kernel,
        mesh=mesh,
        compiler_params=pltpu.CompilerParams(
            needs_layout_passes=False, use_tc_tiling_on_sc=False
        ),
        out_type=[
            jax.ShapeDtypeStruct((NW, F_PER_TILE * N_PAD), jnp.float32),
        ],
        scratch_types=[
            pltpu.VMEM((F_PER_TILE * N_PAD,), jnp.float32),
            pltpu.VMEM((F_PER_TILE * N_PAD,), jnp.float32),
            pltpu.VMEM((N_PAD,), jnp.float32),
            pltpu.VMEM((2, CHUNK_B2), jnp.int32),
            pltpu.VMEM((2, CHUNK_B2), jnp.float32),
            pltpu.SemaphoreType.DMA,
        ],
    )
    def b2(act_hbm, sd_hbm, w_hbm, den_hbm, agg_hbm,
           actc_v, agg_v, den_v, sd_v, wc_v, esem):
        sid = lax.axis_index("s")
        wid = sid * 2 + lax.axis_index("c")
        pltpu.sync_copy(act_hbm.at[wid], actc_v)
        pltpu.sync_copy(den_hbm, den_v)

        zf = jnp.zeros((16,), jnp.float32)

        @plsc.parallel_loop(0, N_PAD // 16, unroll=8)
        def _zero(i):
            den_v[pl.ds(i * 16, 16)] = zf
            for f in range(F_PER_TILE):
                agg_v[pl.ds(f * N_PAD + i * 16, 16)] = zf

        def start_chunk(ci, buf):
            off = ci * CHUNK_B2
            pltpu.async_copy(sd_hbm.at[pl.ds(off, CHUNK_B2)], sd_v.at[buf], esem)
            pltpu.async_copy(w_hbm.at[pl.ds(off, CHUNK_B2)], wc_v.at[buf], esem)

        def wait_chunk(ci, buf):
            off = ci * CHUNK_B2
            pltpu.make_async_copy(
                sd_hbm.at[pl.ds(off, CHUNK_B2)], sd_v.at[buf], esem
            ).wait()
            pltpu.make_async_copy(
                w_hbm.at[pl.ds(off, CHUNK_B2)], wc_v.at[buf], esem
            ).wait()

        start_chunk(0, 0)

        def chunk_body(ci, carry):
            buf = lax.rem(ci, 2)
            wait_chunk(ci, buf)

            @pl.when(ci + 1 < N_CHUNKS_B2)
            def _():
                start_chunk(ci + 1, 1 - buf)

            @plsc.parallel_loop(0, n_grp, unroll=8)
            def _grp(g):
                o = g * 16
                sd = sd_v[buf, pl.ds(o, 16)]
                w16 = wc_v[buf, pl.ds(o, 16)]
                s16 = jnp.bitwise_and(sd, 0xFFFF)
                d16 = jnp.right_shift(sd, 16)
                for f in range(F_PER_TILE):
                    av = plsc.load_gather(actc_v, [s16 + f * N_PAD] if f else [s16])
                    plsc.addupdate_scatter(
                        agg_v, [d16 + f * N_PAD] if f else [d16], av * w16
                    )

            return carry

        lax.fori_loop(0, N_CHUNKS_B2, chunk_body, 0)

        @plsc.parallel_loop(0, N_PAD // 16, unroll=8)
        def _div(i):
            o = i * 16
            dn = den_v[pl.ds(o, 16)]
            for f in range(F_PER_TILE):
                g = agg_v[pl.ds(f * N_PAD + o, 16)]
                agg_v[pl.ds(f * N_PAD + o, 16)] = g / dn

        pltpu.sync_copy(agg_v, agg_hbm.at[wid])

    return b2


def kernel(x, edge_index, W, a):
    x = x.astype(jnp.float32)
    x_pad = jnp.concatenate(
        [x, jnp.zeros((N_PAD - N_NODES, D), jnp.float32)], axis=0
    )
    w_t = W.astype(jnp.float32).T
    a_pair = a.astype(jnp.float32).reshape(2, D)

    loop_ids = jnp.arange(N_NODES, dtype=jnp.int32)
    pad_src = jnp.arange(E_PAD - E_TOT, dtype=jnp.int32) % N_NODES
    pad_dst = jnp.full((E_PAD - E_TOT,), N_NODES, jnp.int32)
    src = jnp.concatenate([edge_index[0].astype(jnp.int32), loop_ids, pad_src])
    dst = jnp.concatenate([edge_index[1].astype(jnp.int32), loop_ids, pad_dst])
    src3 = src.reshape(NW, CHUNKS_B1, B1_CHUNK)
    dst3 = dst.reshape(NW, CHUNKS_B1, B1_CHUNK)
    sd_packed = jnp.bitwise_or(jnp.left_shift(dst, 16), src)

    act, al = _tc_a(x_pad, w_t, a_pair)
    ad = al[0]
    as_ = al[1]

    msgs, w3, _sink, den_part = _make_b1()(act, ad, as_, src3, dst3)
    w_flat = w3.reshape(E_PAD)
    den_sum = jnp.sum(den_part, axis=0)
    den_pad = jnp.where(den_sum == 0.0, 1.0, den_sum)

    act_cols = act.T.reshape(NW, F_PER_TILE * N_PAD)
    agg_cols, = _make_b2()(act_cols, sd_packed, w_flat, den_pad)
    agg_pad = agg_cols.reshape(D, N_PAD).T

    return (
        agg_pad[:N_NODES],
        w_flat[:E_TOT],
        den_pad[:N_NODES],
        msgs,
    )


# speedup vs baseline: 1.1613x; 1.0014x over previous
import functools

import jax
import jax.numpy as jnp
from jax import lax
from jax.experimental import pallas as pl
from jax.experimental.pallas import tpu as pltpu
from jax.experimental.pallas import tpu_sc as plsc

N_NODES = 10000
D = 128
N_PAD = 10112
E_RAW = 320000
E_TOT = E_RAW + N_NODES
NW = 32
NS = 16
B1_CHUNK = 128
CHUNKS_B1 = 82
E_PAD = NW * CHUNKS_B1 * B1_CHUNK
EDGES_PER_TILE = CHUNKS_B1 * B1_CHUNK
FULL_LAST = (E_TOT - (NW - 1) * EDGES_PER_TILE) // B1_CHUNK
TAIL_ROWS = E_TOT - (NW - 1) * EDGES_PER_TILE - FULL_LAST * B1_CHUNK
CHUNK_B2 = 4096
N_CHUNKS_B2 = E_PAD // CHUNK_B2
F_PER_TILE = D // NW


def _tc_a_body(x_ref, wt_ref, ap_ref, act_ref, al_ref):
    act = jnp.dot(x_ref[...], wt_ref[...], preferred_element_type=jnp.float32)
    act_ref[...] = act
    al_ref[...] = lax.dot_general(
        ap_ref[...], act, (((1,), (1,)), ((), ())),
        preferred_element_type=jnp.float32,
        precision=lax.Precision.HIGHEST,
    )


def _tc_a(x_pad, w_t, a_pair):
    blk = N_PAD
    nblk = 1
    return pl.pallas_call(
        _tc_a_body,
        grid=(nblk,),
        in_specs=[
            pl.BlockSpec((blk, D), lambda i: (i, 0)),
            pl.BlockSpec((D, D), lambda i: (0, 0)),
            pl.BlockSpec((2, D), lambda i: (0, 0)),
        ],
        out_specs=[
            pl.BlockSpec((blk, D), lambda i: (i, 0)),
            pl.BlockSpec((2, blk), lambda i: (0, i)),
        ],
        out_shape=[
            jax.ShapeDtypeStruct((N_PAD, D), jnp.float32),
            jax.ShapeDtypeStruct((2, N_PAD), jnp.float32),
        ],
    )(x_pad, w_t, a_pair)


def _make_b1():
    mesh = plsc.VectorSubcoreMesh(
        core_axis_name="c", subcore_axis_name="s", num_cores=2, num_subcores=16
    )
    rows_per_sub = N_PAD // NS

    @functools.partial(
        pl.kernel,
        mesh=mesh,
        compiler_params=pltpu.CompilerParams(
            needs_layout_passes=False, use_tc_tiling_on_sc=False
        ),
        out_type=[
            jax.ShapeDtypeStruct((E_TOT, D), jnp.float32),
            jax.ShapeDtypeStruct((NW, CHUNKS_B1, B1_CHUNK), jnp.float32),
            jax.ShapeDtypeStruct((EDGES_PER_TILE, D), jnp.float32),
            jax.ShapeDtypeStruct((NW, N_PAD), jnp.float32),
        ],
        scratch_types=[
            pltpu.VMEM((CHUNKS_B1, B1_CHUNK), jnp.int32),
            pltpu.VMEM((CHUNKS_B1, B1_CHUNK), jnp.int32),
            pltpu.VMEM((N_PAD,), jnp.float32),
            pltpu.VMEM((N_PAD,), jnp.float32),
            pltpu.VMEM((CHUNKS_B1, B1_CHUNK), jnp.float32),
            pltpu.VMEM((N_PAD,), jnp.float32),
            pltpu.VMEM((4, B1_CHUNK, D), jnp.float32),
            pltpu.SemaphoreType.DMA,
            pltpu.SemaphoreType.DMA,
            pltpu.SemaphoreType.DMA,
            pltpu.SemaphoreType.DMA,
        ],
    )
    def b1(act_hbm, ad_hbm, as_hbm, src_hbm, dst_hbm, msgs_hbm, w_hbm,
           sink_hbm, denp_hbm, src_v, dst_v, ad_v, as_v, w_v, den_v, rows_v,
           gsem0, gsem1, wsem0, wsem1):
        sid = lax.axis_index("s")
        wid = sid * 2 + lax.axis_index("c")
        base = wid * EDGES_PER_TILE
        pltpu.sync_copy(src_hbm.at[wid], src_v)
        pltpu.sync_copy(dst_hbm.at[wid], dst_v)
        pltpu.sync_copy(ad_hbm, ad_v)
        pltpu.sync_copy(as_hbm, as_v)

        zf = jnp.zeros((16,), jnp.float32)

        @plsc.parallel_loop(0, N_PAD // 16, unroll=8)
        def _zero(i):
            den_v[pl.ds(i * 16, 16)] = zf

        pltpu.async_copy(act_hbm.at[src_v.at[0]], rows_v.at[0], gsem0)
        pltpu.async_copy(act_hbm.at[src_v.at[1]], rows_v.at[1], gsem1)

        def step(c, buf, buf2, gsem, wsem):
            pltpu.make_async_copy(
                act_hbm.at[src_v.at[c]], rows_v.at[buf], gsem
            ).wait()

            @pl.when(c >= 2)
            def _():
                pltpu.make_async_copy(
                    rows_v.at[buf], sink_hbm.at[pl.ds(0, B1_CHUNK)], wsem
                ).wait()

            full = jnp.logical_or(wid < NW - 1, c < FULL_LAST)

            @pl.when(full)
            def _():
                pltpu.async_copy(
                    rows_v.at[buf],
                    msgs_hbm.at[pl.ds(base + c * B1_CHUNK, B1_CHUNK)],
                    wsem,
                )

            @pl.when(jnp.logical_not(full))
            def _():
                pltpu.async_copy(
                    rows_v.at[buf],
                    sink_hbm.at[pl.ds(c * B1_CHUNK, B1_CHUNK)],
                    wsem,
                )

            @pl.when(jnp.logical_and(wid == NW - 1, c == FULL_LAST))
            def _():
                pltpu.sync_copy(
                    rows_v.at[buf, pl.ds(0, TAIL_ROWS)],
                    msgs_hbm.at[pl.ds(base + c * B1_CHUNK, TAIL_ROWS)],
                )

            @pl.when(c + 2 < CHUNKS_B1)
            def _():
                pltpu.async_copy(
                    act_hbm.at[src_v.at[c + 2]], rows_v.at[buf2], gsem
                )

            for g in range(B1_CHUNK // 16):
                s16 = src_v[c, pl.ds(g * 16, 16)]
                d16 = dst_v[c, pl.ds(g * 16, 16)]
                t = plsc.load_gather(ad_v, [d16]) + plsc.load_gather(as_v, [s16])
                w16 = jnp.exp(jnp.maximum(t, 0.01 * t))
                w_v[c, pl.ds(g * 16, 16)] = w16
                plsc.addupdate_scatter(den_v, [d16], w16)

        def body(cc, carry):
            half = lax.rem(cc, 2) * 2
            half2 = lax.rem(cc + 1, 2) * 2
            step(cc * 2, half, half2, gsem0, wsem0)
            step(cc * 2 + 1, half + 1, half2 + 1, gsem1, wsem1)
            return carry

        lax.fori_loop(0, CHUNKS_B1 // 2, body, 0)
        pltpu.make_async_copy(
            rows_v.at[0], sink_hbm.at[pl.ds(0, B1_CHUNK)], wsem0
        ).wait()
        pltpu.make_async_copy(
            rows_v.at[0], sink_hbm.at[pl.ds(0, B1_CHUNK)], wsem1
        ).wait()
        pltpu.sync_copy(w_v, w_hbm.at[wid])
        pltpu.sync_copy(den_v, denp_hbm.at[wid])

    return b1


def _make_b2():
    mesh = plsc.VectorSubcoreMesh(
        core_axis_name="c", subcore_axis_name="s", num_cores=2, num_subcores=16
    )
    n_grp = CHUNK_B2 // 16
    e_per_sub = E_PAD // NS

    @functools.partial(
        pl.kernel,
        mesh=mesh,
        compiler_params=pltpu.CompilerParams(
            needs_layout_passes=False, use_tc_tiling_on_sc=False
        ),
        out_type=[
            jax.ShapeDtypeStruct((NW, F_PER_TILE * N_PAD), jnp.float32),
        ],
        scratch_types=[
            pltpu.VMEM((F_PER_TILE * N_PAD,), jnp.float32),
            pltpu.VMEM((F_PER_TILE * N_PAD,), jnp.float32),
            pltpu.VMEM((N_PAD,), jnp.float32),
            pltpu.VMEM((2, CHUNK_B2), jnp.int32),
            pltpu.VMEM((2, CHUNK_B2), jnp.float32),
            pltpu.SemaphoreType.DMA,
        ],
    )
    def b2(act_hbm, sd_hbm, w_hbm, den_hbm, agg_hbm,
           actc_v, agg_v, den_v, sd_v, wc_v, esem):
        sid = lax.axis_index("s")
        wid = sid * 2 + lax.axis_index("c")
        pltpu.sync_copy(act_hbm.at[wid], actc_v)

        zf = jnp.zeros((16,), jnp.float32)

        @plsc.parallel_loop(0, N_PAD // 16, unroll=8)
        def _zero(i):
            den_v[pl.ds(i * 16, 16)] = zf
            for f in range(F_PER_TILE):
                agg_v[pl.ds(f * N_PAD + i * 16, 16)] = zf

        def start_chunk(ci, buf):
            off = ci * CHUNK_B2
            pltpu.async_copy(sd_hbm.at[pl.ds(off, CHUNK_B2)], sd_v.at[buf], esem)
            pltpu.async_copy(w_hbm.at[pl.ds(off, CHUNK_B2)], wc_v.at[buf], esem)

        def wait_chunk(ci, buf):
            off = ci * CHUNK_B2
            pltpu.make_async_copy(
                sd_hbm.at[pl.ds(off, CHUNK_B2)], sd_v.at[buf], esem
            ).wait()
            pltpu.make_async_copy(
                w_hbm.at[pl.ds(off, CHUNK_B2)], wc_v.at[buf], esem
            ).wait()

        start_chunk(0, 0)

        def chunk_body(ci, carry):
            buf = lax.rem(ci, 2)
            wait_chunk(ci, buf)

            @pl.when(ci + 1 < N_CHUNKS_B2)
            def _():
                start_chunk(ci + 1, 1 - buf)

            @plsc.parallel_loop(0, n_grp, unroll=8)
            def _grp(g):
                o = g * 16
                sd = sd_v[buf, pl.ds(o, 16)]
                w16 = wc_v[buf, pl.ds(o, 16)]
                s16 = jnp.bitwise_and(sd, 0xFFFF)
                d16 = jnp.right_shift(sd, 16)
                for f in range(F_PER_TILE):
                    av = plsc.load_gather(actc_v, [s16 + f * N_PAD] if f else [s16])
                    plsc.addupdate_scatter(
                        agg_v, [d16 + f * N_PAD] if f else [d16], av * w16
                    )

            return carry

        lax.fori_loop(0, N_CHUNKS_B2, chunk_body, 0)

        pltpu.sync_copy(den_hbm, den_v)

        @plsc.parallel_loop(0, N_PAD // 16, unroll=8)
        def _div(i):
            o = i * 16
            dn = den_v[pl.ds(o, 16)]
            for f in range(F_PER_TILE):
                g = agg_v[pl.ds(f * N_PAD + o, 16)]
                agg_v[pl.ds(f * N_PAD + o, 16)] = g / dn

        pltpu.sync_copy(agg_v, agg_hbm.at[wid])

    return b2


def kernel(x, edge_index, W, a):
    x = x.astype(jnp.float32)
    x_pad = jnp.concatenate(
        [x, jnp.zeros((N_PAD - N_NODES, D), jnp.float32)], axis=0
    )
    w_t = W.astype(jnp.float32).T
    a_pair = a.astype(jnp.float32).reshape(2, D)

    loop_ids = jnp.arange(N_NODES, dtype=jnp.int32)
    pad_src = jnp.arange(E_PAD - E_TOT, dtype=jnp.int32) % N_NODES
    pad_dst = jnp.full((E_PAD - E_TOT,), N_NODES, jnp.int32)
    src = jnp.concatenate([edge_index[0].astype(jnp.int32), loop_ids, pad_src])
    dst = jnp.concatenate([edge_index[1].astype(jnp.int32), loop_ids, pad_dst])
    src3 = src.reshape(NW, CHUNKS_B1, B1_CHUNK)
    dst3 = dst.reshape(NW, CHUNKS_B1, B1_CHUNK)
    sd_packed = jnp.bitwise_or(jnp.left_shift(dst, 16), src)

    act, al = _tc_a(x_pad, w_t, a_pair)
    ad = al[0]
    as_ = al[1]

    msgs, w3, _sink, den_part = _make_b1()(act, ad, as_, src3, dst3)
    w_flat = w3.reshape(E_PAD)
    den_sum = jnp.sum(den_part, axis=0)
    den_pad = jnp.where(den_sum == 0.0, 1.0, den_sum)

    act_cols = act.T.reshape(NW, F_PER_TILE * N_PAD)
    agg_cols, = _make_b2()(act_cols, sd_packed, w_flat, den_pad)
    agg_pad = agg_cols.reshape(D, N_PAD).T

    return (
        agg_pad[:N_NODES],
        w_flat[:E_TOT],
        den_pad[:N_NODES],
        msgs,
    )
